# baseline (speedup 1.0000x reference)
import functools

import jax
import jax.numpy as jnp
from jax import lax
from jax.experimental import pallas as pl
from jax.experimental.pallas import tpu as pltpu
from jax.experimental.pallas import tpu_sc as plsc

N_NODES = 10000
D_FEAT = 128
N_GRAPHS = 64
CHUNK = 320
HALF = CHUNK // 2
TAIL = N_NODES - 31 * CHUNK
NW = 32


def _seg_body(x_hbm, b_hbm, out_s, out_c, xv, bv, sp, cp, sv, cv, sem0, sem1):
    cid = lax.axis_index("c")
    sid = lax.axis_index("s")
    wid = cid * 16 + sid
    iota = lax.iota(jnp.int32, 16)
    zero16 = jnp.zeros((16,), jnp.float32)
    ones16 = jnp.ones((16,), jnp.float32)

    for l in range(16):
        for gg in range(N_GRAPHS // 16):
            sp[l, pl.ds(gg * 16, 16)] = zero16
            cp[l, pl.ds(gg * 16, 16)] = zero16

    def make_group_body(ngroups):
        def group_body(t, _):
            fb = t * (16 * D_FEAT) + iota * D_FEAT
            bvec = bv[pl.ds(pl.multiple_of(t * 16, 16), 16)]
            accs = [zero16, zero16, zero16, zero16]
            for d in range(D_FEAT):
                idx = fb + (iota ^ d)
                g = plsc.load_gather(xv, [idx])
                accs[d % 4] = accs[d % 4] + g
            acc = (accs[0] + accs[1]) + (accs[2] + accs[3])
            plsc.addupdate_scatter(sp, [iota, bvec], acc)
            plsc.addupdate_scatter(cp, [iota, bvec], ones16)
            return 0
        return group_body

    def process(base, rows):
        half = rows // 2
        c0 = pltpu.async_copy(
            x_hbm.at[pl.ds(base * D_FEAT, half * D_FEAT)],
            xv.at[pl.ds(0, half * D_FEAT)], sem0)
        c1 = pltpu.async_copy(
            x_hbm.at[pl.ds((base + half) * D_FEAT, half * D_FEAT)],
            xv.at[pl.ds(half * D_FEAT, half * D_FEAT)], sem1)
        pltpu.sync_copy(b_hbm.at[pl.ds(base, rows)], bv.at[pl.ds(0, rows)])
        body = make_group_body(rows // 16)
        c0.wait()
        lax.fori_loop(0, half // 16, body, 0)
        c1.wait()
        lax.fori_loop(half // 16, rows // 16, body, 0)

    @pl.when(wid < NW - 1)
    def _():
        process(wid * CHUNK, CHUNK)

    @pl.when(wid == NW - 1)
    def _():
        process((NW - 1) * CHUNK, TAIL)

    for gg in range(N_GRAPHS // 16):
        acc_s = sp[0, pl.ds(gg * 16, 16)]
        acc_c = cp[0, pl.ds(gg * 16, 16)]
        for l in range(1, 16):
            acc_s = acc_s + sp[l, pl.ds(gg * 16, 16)]
            acc_c = acc_c + cp[l, pl.ds(gg * 16, 16)]
        sv[pl.ds(gg * 16, 16)] = acc_s
        cv[pl.ds(gg * 16, 16)] = acc_c
    pltpu.sync_copy(sv, out_s.at[wid])
    pltpu.sync_copy(cv, out_c.at[wid])


_seg = functools.partial(
    pl.kernel,
    out_type=[
        jax.ShapeDtypeStruct((NW, N_GRAPHS), jnp.float32),
        jax.ShapeDtypeStruct((NW, N_GRAPHS), jnp.float32),
    ],
    mesh=plsc.VectorSubcoreMesh(core_axis_name="c", subcore_axis_name="s"),
    compiler_params=pltpu.CompilerParams(needs_layout_passes=False),
    scratch_types=[
        pltpu.VMEM((CHUNK * D_FEAT,), jnp.float32),
        pltpu.VMEM((CHUNK,), jnp.int32),
        pltpu.VMEM((16, N_GRAPHS), jnp.float32),
        pltpu.VMEM((16, N_GRAPHS), jnp.float32),
        pltpu.VMEM((N_GRAPHS,), jnp.float32),
        pltpu.VMEM((N_GRAPHS,), jnp.float32),
        pltpu.SemaphoreType.DMA,
        pltpu.SemaphoreType.DMA,
    ],
)(_seg_body)


def _combine_body(s_ref, c_ref, u_ref, o_ref):
    s = jnp.sum(s_ref[...], axis=0, keepdims=True)
    c = jnp.sum(c_ref[...], axis=0, keepdims=True)
    ones_row = jnp.ones((1, u_ref.shape[1]), jnp.float32)
    us = lax.dot_general(ones_row, u_ref[...],
                         (((1,), (1,)), ((), ())))
    denom = jnp.float32(u_ref.shape[1] + D_FEAT)
    o_ref[...] = (us + s / jnp.maximum(c, 1.0)) / denom


def kernel(x, edge_index, edge_attr, u, batch):
    del edge_index, edge_attr
    b = batch.astype(jnp.int32)
    part_s, part_c = _seg(x.reshape(-1), b)
    out = pl.pallas_call(
        _combine_body,
        out_shape=jax.ShapeDtypeStruct((1, N_GRAPHS), jnp.float32),
    )(part_s, part_c, u)
    return out.reshape(N_GRAPHS)


# speedup vs baseline: 1.4581x; 1.4581x over previous
import functools

import jax
import jax.numpy as jnp
from jax import lax
from jax.experimental import pallas as pl
from jax.experimental.pallas import tpu as pltpu
from jax.experimental.pallas import tpu_sc as plsc

N_NODES = 10000
D_FEAT = 128
N_GRAPHS = 64
CHUNK = 320
HALF = CHUNK // 2
TAIL = N_NODES - 31 * CHUNK
NW = 32


def _seg_body(x_hbm, b_hbm, out_s, out_c, xv, bv, sp, cp, sv, cv, sem0, sem1):
    cid = lax.axis_index("c")
    sid = lax.axis_index("s")
    wid = cid * 16 + sid
    iota = lax.iota(jnp.int32, 16)
    zero16 = jnp.zeros((16,), jnp.float32)
    ones16 = jnp.ones((16,), jnp.float32)

    for l in range(16):
        for gg in range(N_GRAPHS // 16):
            sp[l, pl.ds(gg * 16, 16)] = zero16
            cp[l, pl.ds(gg * 16, 16)] = zero16

    ms = [iota ^ k for k in range(8)]

    def group_body(t, _):
        fb = t * (16 * D_FEAT) + iota * D_FEAT
        bvec = bv[pl.ds(pl.multiple_of(t * 16, 16), 16)]

        def dstep(j, carry):
            a0, a1 = carry
            dsp = jnp.full((16,), j * 8, jnp.int32)
            for k in range(8):
                idx = fb + (ms[k] ^ dsp)
                g = plsc.load_gather(xv, [idx])
                if k % 2 == 0:
                    a0 = a0 + g
                else:
                    a1 = a1 + g
            return (a0, a1)

        a0, a1 = lax.fori_loop(0, D_FEAT // 8, dstep, (zero16, zero16))
        acc = a0 + a1
        plsc.addupdate_scatter(sp, [iota, bvec], acc)
        plsc.addupdate_scatter(cp, [iota, bvec], ones16)
        return 0

    def process(base, rows):
        half = rows // 2
        c0 = pltpu.async_copy(
            x_hbm.at[pl.ds(base * D_FEAT, half * D_FEAT)],
            xv.at[pl.ds(0, half * D_FEAT)], sem0)
        c1 = pltpu.async_copy(
            x_hbm.at[pl.ds((base + half) * D_FEAT, half * D_FEAT)],
            xv.at[pl.ds(half * D_FEAT, half * D_FEAT)], sem1)
        pltpu.sync_copy(b_hbm.at[pl.ds(base, rows)], bv.at[pl.ds(0, rows)])
        c0.wait()
        lax.fori_loop(0, half // 16, group_body, 0)
        c1.wait()
        lax.fori_loop(half // 16, rows // 16, group_body, 0)

    @pl.when(wid < NW - 1)
    def _():
        process(wid * CHUNK, CHUNK)

    @pl.when(wid == NW - 1)
    def _():
        process((NW - 1) * CHUNK, TAIL)

    for gg in range(N_GRAPHS // 16):
        acc_s = sp[0, pl.ds(gg * 16, 16)]
        acc_c = cp[0, pl.ds(gg * 16, 16)]
        for l in range(1, 16):
            acc_s = acc_s + sp[l, pl.ds(gg * 16, 16)]
            acc_c = acc_c + cp[l, pl.ds(gg * 16, 16)]
        sv[pl.ds(gg * 16, 16)] = acc_s
        cv[pl.ds(gg * 16, 16)] = acc_c
    pltpu.sync_copy(sv, out_s.at[wid])
    pltpu.sync_copy(cv, out_c.at[wid])


_seg = functools.partial(
    pl.kernel,
    out_type=[
        jax.ShapeDtypeStruct((NW, N_GRAPHS), jnp.float32),
        jax.ShapeDtypeStruct((NW, N_GRAPHS), jnp.float32),
    ],
    mesh=plsc.VectorSubcoreMesh(core_axis_name="c", subcore_axis_name="s"),
    compiler_params=pltpu.CompilerParams(needs_layout_passes=False),
    scratch_types=[
        pltpu.VMEM((CHUNK * D_FEAT,), jnp.float32),
        pltpu.VMEM((CHUNK,), jnp.int32),
        pltpu.VMEM((16, N_GRAPHS), jnp.float32),
        pltpu.VMEM((16, N_GRAPHS), jnp.float32),
        pltpu.VMEM((N_GRAPHS,), jnp.float32),
        pltpu.VMEM((N_GRAPHS,), jnp.float32),
        pltpu.SemaphoreType.DMA,
        pltpu.SemaphoreType.DMA,
    ],
)(_seg_body)


def _combine_body(s_ref, c_ref, u_ref, o_ref):
    s = jnp.sum(s_ref[...], axis=0, keepdims=True)
    c = jnp.sum(c_ref[...], axis=0, keepdims=True)
    ones_row = jnp.ones((1, u_ref.shape[1]), jnp.float32)
    us = lax.dot_general(ones_row, u_ref[...],
                         (((1,), (1,)), ((), ())))
    denom = jnp.float32(u_ref.shape[1] + D_FEAT)
    o_ref[...] = (us + s / jnp.maximum(c, 1.0)) / denom


def kernel(x, edge_index, edge_attr, u, batch):
    del edge_index, edge_attr
    b = batch.astype(jnp.int32)
    part_s, part_c = _seg(x.reshape(-1), b)
    out = pl.pallas_call(
        _combine_body,
        out_shape=jax.ShapeDtypeStruct((1, N_GRAPHS), jnp.float32),
    )(part_s, part_c, u)
    return out.reshape(N_GRAPHS)
